# baseline (speedup 1.0000x reference)
import functools

import jax
import jax.numpy as jnp
from jax import lax
from jax.experimental import pallas as pl
from jax.experimental.pallas import tpu as pltpu
from jax.experimental.pallas import tpu_sc as plsc

D = 64
SCALE = 8.0
LANES = 16


def kernel(x, emb):
    B, S = x.shape
    N = B * S
    info = plsc.get_sparse_core_info()
    NC, NS = info.num_cores, info.num_subcores
    NW = NC * NS
    n_per_w = N // NW
    C = 512
    n_chunks = n_per_w // C

    mesh = plsc.VectorSubcoreMesh(core_axis_name="c", subcore_axis_name="s")

    @functools.partial(
        pl.kernel,
        mesh=mesh,
        out_type=jax.ShapeDtypeStruct((N, D), jnp.float32),
        scratch_types=[
            pltpu.VMEM((C,), jnp.int32),
            pltpu.VMEM((C, D), jnp.float32),
            pltpu.SemaphoreType.DMA,
        ],
        compiler_params=pltpu.CompilerParams(use_tc_tiling_on_sc=False),
    )
    def emb_kernel(idx_hbm, table_hbm, out_hbm, idx_v, rows_v, sem):
        wid = lax.axis_index("s") * NC + lax.axis_index("c")
        base = wid * n_per_w

        def chunk_body(c, carry):
            off = base + c * C
            pltpu.sync_copy(idx_hbm.at[pl.ds(off, C)], idx_v)
            pltpu.async_copy(table_hbm.at[idx_v], rows_v, sem).wait()

            def scale_row(j, carry2):
                for t in range(D // LANES):
                    v = rows_v[j, pl.ds(t * LANES, LANES)]
                    rows_v[j, pl.ds(t * LANES, LANES)] = v * SCALE
                return carry2

            lax.fori_loop(0, C, scale_row, 0, unroll=False)
            pltpu.sync_copy(rows_v, out_hbm.at[pl.ds(off, C)])
            return carry

        lax.fori_loop(0, n_chunks, chunk_body, 0, unroll=False)

    y = emb_kernel(x.reshape(N), emb)
    return y.reshape(B, S, D)


# speedup vs baseline: 1.1186x; 1.1186x over previous
import functools

import jax
import jax.numpy as jnp
from jax import lax
from jax.experimental import pallas as pl
from jax.experimental.pallas import tpu as pltpu
from jax.experimental.pallas import tpu_sc as plsc

D = 64
SCALE = 8.0
LANES = 16


def kernel(x, emb):
    B, S = x.shape
    N = B * S
    info = plsc.get_sparse_core_info()
    NC, NS = info.num_cores, info.num_subcores
    NW = NC * NS
    n_per_w = N // NW
    C = 512
    n_chunks = n_per_w // C
    assert n_chunks % 2 == 0

    mesh = plsc.VectorSubcoreMesh(core_axis_name="c", subcore_axis_name="s")

    @functools.partial(
        pl.kernel,
        mesh=mesh,
        out_type=jax.ShapeDtypeStruct((N, D), jnp.float32),
        scratch_types=[
            pltpu.VMEM((C,), jnp.int32),
            pltpu.VMEM((C,), jnp.int32),
            pltpu.VMEM((C, D), jnp.float32),
            pltpu.VMEM((C, D), jnp.float32),
            pltpu.SemaphoreType.DMA,
            pltpu.SemaphoreType.DMA,
            pltpu.SemaphoreType.DMA,
            pltpu.SemaphoreType.DMA,
        ],
        compiler_params=pltpu.CompilerParams(use_tc_tiling_on_sc=False),
    )
    def emb_kernel(idx_hbm, table_hbm, out_hbm, idx0, idx1, rows0, rows1,
                   gsem0, gsem1, osem0, osem1):
        wid = lax.axis_index("s") * NC + lax.axis_index("c")
        base = wid * n_per_w
        idx_v = (idx0, idx1)
        rows_v = (rows0, rows1)
        gsem = (gsem0, gsem1)
        osem = (osem0, osem1)

        def start_gather(c, b):
            off = base + c * C
            pltpu.sync_copy(idx_hbm.at[pl.ds(off, C)], idx_v[b])
            pltpu.async_copy(table_hbm.at[idx_v[b]], rows_v[b], gsem[b])

        def scale_rows(rows):
            def scale_row(j, carry):
                for t in range(D // LANES):
                    v = rows[j, pl.ds(t * LANES, LANES)]
                    rows[j, pl.ds(t * LANES, LANES)] = v * SCALE
                return carry

            lax.fori_loop(0, C, scale_row, 0, unroll=4)

        def wait_gather(b):
            pltpu.make_async_copy(table_hbm.at[idx_v[b]], rows_v[b],
                                  gsem[b]).wait()

        def wait_out(b):
            pltpu.make_async_copy(rows_v[b], out_hbm.at[pl.ds(0, C)],
                                  osem[b]).wait()

        def writeback(c, b):
            off = base + c * C
            pltpu.async_copy(rows_v[b], out_hbm.at[pl.ds(off, C)], osem[b])

        start_gather(0, 0)

        def pair_body(g, carry):
            wait_gather(0)

            @pl.when(g >= 1)
            def _():
                wait_out(1)

            start_gather(2 * g + 1, 1)
            scale_rows(rows0)
            writeback(2 * g, 0)

            wait_gather(1)
            wait_out(0)

            @pl.when(2 * g + 2 < n_chunks)
            def _():
                start_gather(2 * g + 2, 0)

            scale_rows(rows1)
            writeback(2 * g + 1, 1)
            return carry

        lax.fori_loop(0, n_chunks // 2, pair_body, 0, unroll=False)

        wait_out(1)

    y = emb_kernel(x.reshape(N), emb)
    return y.reshape(B, S, D)
